# baseline (speedup 1.0000x reference)
import functools

import jax
import jax.numpy as jnp
from jax import lax
from jax.experimental import pallas as pl
from jax.experimental.pallas import tpu as pltpu
from jax.experimental.pallas import tpu_sc as plsc

NE = 64
KTOP = 8
DD = 8
NC = 2
NS = 16
LL = 16


def _logits_body(hT_ref, gw_ref, gb_ref, out_ref):
    logits = jax.lax.dot_general(
        gw_ref[...], hT_ref[...], (((1,), (0,)), ((), ())),
        preferred_element_type=jnp.float32) + gb_ref[...]
    out_ref[...] = logits


def _combine_body(hT_ref, gw_ref, gb_ref, wT_ref, bT_ref, m8_ref, out_ref):
    h = hT_ref[...]
    logits = jax.lax.dot_general(
        gw_ref[...], h, (((1,), (0,)), ((), ())),
        preferred_element_type=jnp.float32) + gb_ref[...]
    sel_acc = jnp.where(logits >= m8_ref[...], 1.0, 0.0).astype(jnp.bfloat16)
    cw = jax.lax.dot_general(
        wT_ref[...].astype(jnp.bfloat16), sel_acc, (((1,), (0,)), ((), ())),
        preferred_element_type=jnp.float32)
    hh = jnp.concatenate([h] * DD, axis=0)
    prod = (hh * cw).astype(jnp.bfloat16)
    r8 = jax.lax.broadcasted_iota(jnp.int32, (DD, NE), 0)
    c64 = jax.lax.broadcasted_iota(jnp.int32, (DD, NE), 1)
    sel_mat = jnp.where(c64 // DD == r8, 1.0, 0.0).astype(jnp.bfloat16)
    out = jax.lax.dot_general(
        sel_mat, prod, (((1,), (0,)), ((), ())),
        preferred_element_type=jnp.float32)
    cb = jax.lax.dot_general(
        bT_ref[...].astype(jnp.bfloat16), sel_acc, (((1,), (0,)), ((), ())),
        preferred_element_type=jnp.float32)
    out_ref[...] = out + cb


def _make_sc_threshold(T):
    TPW = T // (NC * NS)
    mesh = plsc.VectorSubcoreMesh(core_axis_name="c", subcore_axis_name="s")

    @functools.partial(
        pl.kernel, mesh=mesh,
        out_type=jax.ShapeDtypeStruct((T,), jnp.float32),
        scratch_types=[
            pltpu.VMEM((NE, TPW), jnp.float32),
            pltpu.VMEM((TPW,), jnp.float32),
            pltpu.SemaphoreType.DMA,
        ],
    )
    def sc_threshold(logits_hbm, out_hbm, buf, m8buf, sem):
        wid = lax.axis_index("s") * NC + lax.axis_index("c")
        base = wid * TPW
        descs = [
            pltpu.async_copy(logits_hbm.at[e, pl.ds(base, TPW)],
                             buf.at[e], sem)
            for e in range(NE)
        ]
        for d in descs:
            d.wait()

        GI = 4

        def group_body(g, carry):
            t = [[jnp.full((LL,), -jnp.inf, jnp.float32)
                  for _ in range(KTOP)] for _ in range(GI)]
            for e in range(NE):
                for u in range(GI):
                    c = buf[e, pl.ds((GI * g + u) * LL, LL)]
                    for i in range(KTOP):
                        hi = jnp.maximum(t[u][i], c)
                        if i < KTOP - 1:
                            c = jnp.minimum(t[u][i], c)
                        t[u][i] = hi
            for u in range(GI):
                m8buf[pl.ds((GI * g + u) * LL, LL)] = t[u][KTOP - 1]
            return carry

        lax.fori_loop(0, TPW // (LL * GI), group_body, 0)
        pltpu.sync_copy(m8buf, out_hbm.at[pl.ds(base, TPW)])

    return sc_threshold


@functools.partial(jax.jit, static_argnames=("interpret",))
def kernel(hidden_states, gate_w, gate_b, expert_ws, expert_bs,
           interpret=False):
    B, S, D = hidden_states.shape
    T = B * S
    Tb = 16384
    hT = hidden_states.reshape(T, D).T
    wT = expert_ws.reshape(NE, NE).T
    bT = expert_bs.T
    gb = gate_b.reshape(NE, 1)

    logits = pl.pallas_call(
        _logits_body,
        grid=(T // Tb,),
        in_specs=[
            pl.BlockSpec((D, Tb), lambda i: (0, i)),
            pl.BlockSpec((NE, D), lambda i: (0, 0)),
            pl.BlockSpec((NE, 1), lambda i: (0, 0)),
        ],
        out_specs=pl.BlockSpec((NE, Tb), lambda i: (0, i)),
        out_shape=jax.ShapeDtypeStruct((NE, T), jnp.float32),
        interpret=interpret,
    )(hT, gate_w, gb)

    m8 = _make_sc_threshold(T)(logits).reshape(1, T)

    out = pl.pallas_call(
        _combine_body,
        grid=(T // Tb,),
        in_specs=[
            pl.BlockSpec((D, Tb), lambda i: (0, i)),
            pl.BlockSpec((NE, D), lambda i: (0, 0)),
            pl.BlockSpec((NE, 1), lambda i: (0, 0)),
            pl.BlockSpec((NE, NE), lambda i: (0, 0)),
            pl.BlockSpec((D, NE), lambda i: (0, 0)),
            pl.BlockSpec((1, Tb), lambda i: (0, i)),
        ],
        out_specs=pl.BlockSpec((D, Tb), lambda i: (0, i)),
        out_shape=jax.ShapeDtypeStruct((D, T), jnp.float32),
        interpret=interpret,
    )(hT, gate_w, gb, wT, bT, m8)
    return out.T.reshape(B, S, D)


# speedup vs baseline: 4.0447x; 4.0447x over previous
import functools

import jax
import jax.numpy as jnp
from jax.experimental import pallas as pl

NE = 64
KTOP = 8
DD = 8


def _moe_body(hT_ref, gw_ref, gb_ref, wT_ref, bT_ref, out_ref):
    h = hT_ref[...]
    gw = gw_ref[...]
    logits = jax.lax.dot_general(
        gw, h, (((1,), (0,)), ((), ())),
        preferred_element_type=jnp.float32) + gb_ref[...]

    rows = [logits[DD * i:DD * (i + 1), :] for i in range(DD)]
    net = [(0, 1), (2, 3), (4, 5), (6, 7),
           (0, 2), (1, 3), (4, 6), (5, 7),
           (1, 2), (5, 6), (0, 4), (1, 5),
           (2, 6), (3, 7), (2, 4), (3, 5),
           (1, 2), (3, 4), (5, 6)]
    for a, b in net:
        hi = jnp.maximum(rows[a], rows[b])
        lo = jnp.minimum(rows[a], rows[b])
        rows[a], rows[b] = hi, lo
    for j in range(1, KTOP):
        m = jnp.max(rows[0], axis=0, keepdims=True)
        sel = rows[0] == m
        for i in range(DD - j):
            rows[i] = jnp.where(sel, rows[i + 1], rows[i])
    m8 = jnp.max(rows[0], axis=0, keepdims=True)
    sel_acc = jnp.where(logits >= m8, 1.0, 0.0).astype(jnp.bfloat16)

    cw = jax.lax.dot_general(
        wT_ref[...].astype(jnp.bfloat16), sel_acc, (((1,), (0,)), ((), ())),
        preferred_element_type=jnp.float32)

    hh = jnp.concatenate([h] * DD, axis=0)
    prod = (hh * cw).astype(jnp.bfloat16)

    r8 = jax.lax.broadcasted_iota(jnp.int32, (DD, NE), 0)
    c64 = jax.lax.broadcasted_iota(jnp.int32, (DD, NE), 1)
    sel_mat = jnp.where(c64 // DD == r8, 1.0, 0.0).astype(jnp.bfloat16)
    out = jax.lax.dot_general(
        sel_mat, prod, (((1,), (0,)), ((), ())),
        preferred_element_type=jnp.float32)
    cb = jax.lax.dot_general(
        bT_ref[...].astype(jnp.bfloat16), sel_acc, (((1,), (0,)), ((), ())),
        preferred_element_type=jnp.float32)
    out_ref[...] = out + cb


@functools.partial(jax.jit, static_argnames=("interpret",))
def kernel(hidden_states, gate_w, gate_b, expert_ws, expert_bs,
           interpret=False):
    B, S, D = hidden_states.shape
    T = B * S
    Tb = 16384
    hT = hidden_states.reshape(T, D).T
    wT = expert_ws.reshape(NE, NE).T
    bT = expert_bs.T
    gb = gate_b.reshape(NE, 1)

    out = pl.pallas_call(
        _moe_body,
        grid=(T // Tb,),
        in_specs=[
            pl.BlockSpec((D, Tb), lambda i: (0, i)),
            pl.BlockSpec((NE, D), lambda i: (0, 0)),
            pl.BlockSpec((NE, 1), lambda i: (0, 0)),
            pl.BlockSpec((NE, NE), lambda i: (0, 0)),
            pl.BlockSpec((D, NE), lambda i: (0, 0)),
        ],
        out_specs=pl.BlockSpec((D, Tb), lambda i: (0, i)),
        out_shape=jax.ShapeDtypeStruct((D, T), jnp.float32),
        interpret=interpret,
    )(hT, gate_w, gb, wT, bT)
    return out.T.reshape(B, S, D)


# speedup vs baseline: 4.0607x; 1.0039x over previous
import jax
import jax.numpy as jnp
from jax.experimental import pallas as pl

NE = 64
KTOP = 8
DD = 8


def _moe_body(hT_ref, gw_ref, gb_ref, wT_ref, bT_ref, out_ref):
    h = hT_ref[...]
    gw = gw_ref[...]
    logits = jax.lax.dot_general(
        gw, h, (((1,), (0,)), ((), ())),
        preferred_element_type=jnp.float32) + gb_ref[...]

    rows = [logits[DD * i:DD * (i + 1), :] for i in range(DD)]
    net = [(0, 1), (2, 3), (4, 5), (6, 7),
           (0, 2), (1, 3), (4, 6), (5, 7),
           (1, 2), (5, 6), (0, 4), (1, 5),
           (2, 6), (3, 7), (2, 4), (3, 5),
           (1, 2), (3, 4), (5, 6)]
    for a, b in net:
        hi = jnp.maximum(rows[a], rows[b])
        lo = jnp.minimum(rows[a], rows[b])
        rows[a], rows[b] = hi, lo
    for j in range(1, KTOP):
        m = jnp.max(rows[0], axis=0, keepdims=True)
        sel = rows[0] == m
        for i in range(DD - j):
            rows[i] = jnp.where(sel, rows[i + 1], rows[i])
    m8 = jnp.max(rows[0], axis=0, keepdims=True)
    sel_acc = jnp.where(logits >= m8, 1.0, 0.0).astype(jnp.bfloat16)

    cw = jax.lax.dot_general(
        wT_ref[...].astype(jnp.bfloat16), sel_acc, (((1,), (0,)), ((), ())),
        preferred_element_type=jnp.float32)

    hh = jnp.concatenate([h] * DD, axis=0)
    prod = (hh * cw).astype(jnp.bfloat16)

    r8 = jax.lax.broadcasted_iota(jnp.int32, (DD, NE), 0)
    c64 = jax.lax.broadcasted_iota(jnp.int32, (DD, NE), 1)
    sel_mat = jnp.where(c64 // DD == r8, 1.0, 0.0).astype(jnp.bfloat16)
    out = jax.lax.dot_general(
        sel_mat, prod, (((1,), (0,)), ((), ())),
        preferred_element_type=jnp.float32)
    cb = jax.lax.dot_general(
        bT_ref[...].astype(jnp.bfloat16), sel_acc, (((1,), (0,)), ((), ())),
        preferred_element_type=jnp.float32)
    out_ref[...] = out + cb


@jax.jit
def kernel(hidden_states, gate_w, gate_b, expert_ws, expert_bs):
    B, S, D = hidden_states.shape
    T = B * S
    Tb = 16384
    hT = hidden_states.reshape(T, D).T
    wT = expert_ws.reshape(NE, NE).T
    bT = expert_bs.T
    gb = gate_b.reshape(NE, 1)

    out = pl.pallas_call(
        _moe_body,
        grid=(T // Tb,),
        in_specs=[
            pl.BlockSpec((D, Tb), lambda i: (0, i)),
            pl.BlockSpec((NE, D), lambda i: (0, 0)),
            pl.BlockSpec((NE, 1), lambda i: (0, 0)),
            pl.BlockSpec((NE, NE), lambda i: (0, 0)),
            pl.BlockSpec((D, NE), lambda i: (0, 0)),
        ],
        out_specs=pl.BlockSpec((D, Tb), lambda i: (0, i)),
        out_shape=jax.ShapeDtypeStruct((D, T), jnp.float32),
    )(hT, gate_w, gb, wT, bT)
    return out.T.reshape(B, S, D)
